# Initial kernel scaffold; baseline (speedup 1.0000x reference)
import jax
import jax.numpy as jnp
from jax.experimental import pallas as pl


def kernel(edge_index, edge_batch, edge_relations, question_tokens, node_tokens, node_ptr, edge_ptr, start_node_locals, start_ptr, answer_node_locals, answer_ptr, dummy_mask, node_batch, node_in_degree, node_is_start, node_is_answer):
    raise NotImplementedError("write your pallas kernel here")


# speedup vs baseline: 1.7271x; 1.7271x over previous
import functools

import jax
import jax.numpy as jnp
from jax import lax
from jax.experimental import pallas as pl
from jax.experimental.pallas import tpu as pltpu
from jax.experimental.pallas import tpu_sc as plsc

MAX_STEPS = 8
STOP_RELATION = -1
DIRECTION_FORWARD = 0

_LANES = 16


@functools.lru_cache(maxsize=None)
def _make_sc_segmin(B, per_n, sentinel):
    mesh = plsc.VectorSubcoreMesh(core_axis_name="c", subcore_axis_name="s")

    @functools.partial(
        pl.kernel,
        mesh=mesh,
        compiler_params=pltpu.CompilerParams(needs_layout_passes=False),
        out_type=(
            jax.ShapeDtypeStruct((B, _LANES), jnp.int32),
            jax.ShapeDtypeStruct((B, _LANES), jnp.int32),
            jax.ShapeDtypeStruct((B, _LANES), jnp.int32),
            jax.ShapeDtypeStruct((B, _LANES), jnp.int32),
        ),
        scratch_types=[
            pltpu.VMEM((per_n,), jnp.int32),
            pltpu.VMEM((per_n,), jnp.int32),
            pltpu.VMEM((_LANES,), jnp.int32),
            pltpu.VMEM((_LANES,), jnp.int32),
        ],
    )
    def sc_segmin(start_hbm, answer_hbm, extra_hbm,
                  minl_hbm, hits_hbm, ans_hbm, done_hbm,
                  s_v, a_v, stage_v, extra_v):
        c = lax.axis_index("c")
        s = lax.axis_index("s")

        @pl.when(c == 0)
        def _scan():
            base = s * per_n
            pltpu.sync_copy(start_hbm.at[pl.ds(base, per_n)], s_v)
            pltpu.sync_copy(answer_hbm.at[pl.ds(base, per_n)], a_v)
            pltpu.sync_copy(extra_hbm, extra_v)

            def body(i, acc):
                off = i * _LANES
                vs = s_v[pl.ds(off, _LANES)]
                va = a_v[pl.ds(off, _LANES)]
                idx = lax.iota(jnp.int32, _LANES) + off
                hit = (vs > 0) & (va > 0)
                return jnp.minimum(acc, jnp.where(hit, idx, sentinel))

            acc = lax.fori_loop(
                0, per_n // _LANES, body,
                jnp.full((_LANES,), sentinel, jnp.int32))

            dnums = lax.GatherDimensionNumbers(
                offset_dims=(), collapsed_slice_dims=(0,),
                start_index_map=(0,))
            for off in (8, 4, 2, 1):
                perm = (lax.iota(jnp.int32, _LANES) + off) & (_LANES - 1)
                rot = lax.gather(
                    acc, perm[:, None], dimension_numbers=dnums,
                    slice_sizes=(1,),
                    mode=lax.GatherScatterMode.PROMISE_IN_BOUNDS)
                acc = jnp.minimum(acc, rot)

            hit_mask = acc != sentinel
            hits_i = hit_mask.astype(jnp.int32)
            ansh = jnp.where(hit_mask, acc, -1)
            ev = extra_v[...]
            perm_s = jnp.broadcast_to(s, (_LANES,)).astype(jnp.int32)
            extra_b = lax.gather(
                ev, perm_s[:, None], dimension_numbers=dnums,
                slice_sizes=(1,),
                mode=lax.GatherScatterMode.PROMISE_IN_BOUNDS)
            done_i = jnp.maximum(hits_i, extra_b)

            stage_v[...] = acc
            pltpu.sync_copy(stage_v, minl_hbm.at[s])
            stage_v[...] = hits_i
            pltpu.sync_copy(stage_v, hits_hbm.at[s])
            stage_v[...] = ansh
            pltpu.sync_copy(stage_v, ans_hbm.at[s])
            stage_v[...] = done_i
            pltpu.sync_copy(stage_v, done_hbm.at[s])

    return sc_segmin


def kernel(edge_index, edge_batch, edge_relations, question_tokens, node_tokens,
           node_ptr, edge_ptr, start_node_locals, start_ptr,
           answer_node_locals, answer_ptr, dummy_mask,
           node_batch, node_in_degree, node_is_start, node_is_answer):
    B = int(node_ptr.shape[0]) - 1
    N = int(node_is_start.shape[0])
    E = int(edge_index.shape[1])
    per_n = N // B
    sentinel = N + 1

    question_tokens = question_tokens.astype(jnp.float32)
    node_tokens = node_tokens.astype(jnp.float32)

    start_i = node_is_start.astype(jnp.int32)
    answer_i = node_is_answer.astype(jnp.int32)
    start_counts = start_ptr[1:] - start_ptr[:-1]
    extra_i = ((start_counts == 0) | dummy_mask).astype(jnp.int32)

    sc_segmin = _make_sc_segmin(B, per_n, sentinel)
    _minl16, hits16, ans16, done16 = sc_segmin(start_i, answer_i, extra_i)

    answer_hits = hits16[:, 0].astype(bool)
    done = done16[:, 0].astype(bool)
    answer_node_hit = ans16[:, 0]
    start_node_hit = answer_node_hit

    active_nodes = node_is_start
    visited_nodes = node_is_start
    used_edge_mask = jnp.zeros((E,), dtype=bool)
    actions = jnp.full((B, MAX_STEPS + 1), STOP_RELATION, dtype=jnp.int32)
    directions = jnp.full((B, MAX_STEPS + 1), DIRECTION_FORWARD, dtype=jnp.int32)
    step_counts = jnp.zeros((B,), dtype=jnp.int32)

    return (active_nodes, visited_nodes, used_edge_mask, actions, directions,
            done, step_counts, answer_hits, answer_node_hit, start_node_hit,
            node_tokens, question_tokens)
